# baseline (speedup 1.0000x reference)
import functools

import jax
import jax.numpy as jnp
from jax import lax
from jax.experimental import pallas as pl

_B, _Q, _C1 = 32, 1000, 4
_T, _P = 100, 50
_PAIRS = _B * _T
_NQ = _B * _Q
_W = 128
_NW = _PAIRS // _W


def _loss_kernel(logits_ref, labels_ref, s_ref, t_ref, out_ref):
    lg = logits_ref[...]
    m = jnp.max(lg, axis=0, keepdims=True)
    lse = jnp.log(jnp.sum(jnp.exp(lg - m), axis=0, keepdims=True)) + m
    lab = labels_ref[...]
    cls = lax.broadcasted_iota(jnp.int32, (_C1, _NQ), 0)
    matched = jnp.sum(jnp.where(cls == lab, lg, 0.0), axis=0, keepdims=True)
    ce = jnp.sum(lse - matched) / _NQ

    poly = 0.0
    for w in range(_NW):
        sl = slice(w * _W, (w + 1) * _W)
        sx = s_ref[0, :, sl]
        sy = s_ref[1, :, sl]
        tx = t_ref[0, :, sl]
        ty = t_ref[1, :, sl]
        macc = None
        acc1 = None
        for j in range(_P):
            txj = tx[j:j + 1]
            tyj = ty[j:j + 1]
            d = jnp.abs(sx - txj) + jnp.abs(sy - tyj)
            macc = d if macc is None else jnp.minimum(macc, d)
            cmin = jnp.min(d, axis=0, keepdims=True)
            acc1 = cmin if acc1 is None else acc1 + cmin
        per_t = acc1 + jnp.sum(macc, axis=0, keepdims=True)
        poly = poly + jnp.sum(per_t)
    poly = poly * (0.5 / (_PAIRS * _P))

    sdx = s_ref[0, _P - 1, :] - s_ref[0, 0, :]
    sdy = s_ref[1, _P - 1, :] - s_ref[1, 0, :]
    tdx = t_ref[0, _P - 1, :] - t_ref[0, 0, :]
    tdy = t_ref[1, _P - 1, :] - t_ref[1, 0, :]
    sn = jnp.sqrt(sdx * sdx + sdy * sdy) + 1e-6
    tn = jnp.sqrt(tdx * tdx + tdy * tdy) + 1e-6
    cos = (sdx * tdx + sdy * tdy) / (sn * tn)
    direc = jnp.sum(1.0 - cos) / _PAIRS

    idx = lax.broadcasted_iota(jnp.int32, (3,), 0)
    out_ref[...] = (jnp.where(idx == 0, ce, 0.0)
                    + jnp.where(idx == 1, poly, 0.0)
                    + jnp.where(idx == 2, direc, 0.0))


@jax.jit
def kernel(pred_logits, pred_polylines, tgt_labels, tgt_polylines):
    B, Q, C1 = pred_logits.shape
    T = tgt_labels.shape[1]
    P = pred_polylines.shape[2]

    logits_t = jnp.transpose(pred_logits.reshape(B * Q, C1), (1, 0))
    labels_full = jnp.concatenate(
        [tgt_labels.astype(jnp.int32),
         jnp.full((B, Q - T), C1 - 1, dtype=jnp.int32)], axis=1)
    labels_full = labels_full.reshape(1, B * Q)
    s_t = jnp.transpose(pred_polylines[:, :T], (3, 2, 0, 1)).reshape(2, P, B * T)
    t_t = jnp.transpose(tgt_polylines, (3, 2, 0, 1)).reshape(2, P, B * T)

    out = pl.pallas_call(
        _loss_kernel,
        out_shape=jax.ShapeDtypeStruct((3,), jnp.float32),
    )(logits_t, labels_full, s_t, t_t)
    return out
